# Initial kernel scaffold; baseline (speedup 1.0000x reference)
import jax
import jax.numpy as jnp
from jax.experimental import pallas as pl


def kernel(x, edge_index, W_m0, b_m0, W_v0, b_v0, W_m1, b_m1, W_v1, b_v1):
    raise NotImplementedError("write your pallas kernel here")


# speedup vs baseline: 35.6530x; 35.6530x over previous
import functools

import jax
import jax.numpy as jnp
from jax import lax
from jax.experimental import pallas as pl
from jax.experimental.pallas import tpu as pltpu
from jax.experimental.pallas import tpu_sc as plsc

N = 10000
E = 320000
D = 128

NC = 2
NS = 16
NW = NC * NS
EPW = E // NW
KB = 100
NB = EPW // KB
NP = 10240
RPT = NP // NS
FC = 64
BLK = 2000
GRID = N // BLK


def _make_degree():
    mesh = plsc.VectorSubcoreMesh(core_axis_name="c", subcore_axis_name="s")

    @functools.partial(
        pl.kernel,
        out_type=(
            jax.ShapeDtypeStruct((NP,), jnp.float32),
            jax.ShapeDtypeStruct((NP,), jnp.float32),
        ),
        mesh=mesh,
        scratch_types=[
            pltpu.VMEM((NB, KB), jnp.int32),
            pltpu.VMEM((128,), jnp.float32),
            pltpu.VMEM((2048,), jnp.float32),
            pltpu.VMEM_SHARED((NP,), jnp.float32),
        ],
    )
    def deg_kernel(row_hbm, out0, out1, row_v, ones_v, zbuf, acc):
        c = lax.axis_index("c")
        s = lax.axis_index("s")
        w = c * NS + s

        def zfill(j, carry):
            zbuf[pl.ds(j * 16, 16)] = jnp.zeros((16,), jnp.float32)
            return carry

        lax.fori_loop(0, 128, zfill, 0)

        @pl.when(s < 5)
        def _():
            pltpu.sync_copy(zbuf, acc.at[pl.ds(s * 2048, 2048)])

        for k in range(8):
            ones_v[pl.ds(k * 16, 16)] = jnp.ones((16,), jnp.float32)
        pltpu.sync_copy(row_hbm.at[w], row_v)
        plsc.subcore_barrier()

        def body(j, carry):
            pltpu.sync_copy(ones_v.at[pl.ds(0, KB)], acc.at[row_v.at[j]],
                            add=True)
            return carry

        lax.fori_loop(0, NB, body, 0)
        plsc.subcore_barrier()

        @pl.when(s < 5)
        def _():
            sl = pl.ds(s * 2048, 2048)
            pltpu.sync_copy(acc.at[sl], zbuf)

            @pl.when(c == 0)
            def _():
                pltpu.sync_copy(zbuf, out0.at[sl])

            @pl.when(c == 1)
            def _():
                pltpu.sync_copy(zbuf, out1.at[sl])

    return deg_kernel


_degree = _make_degree()


def _make_spmm():
    mesh = plsc.VectorSubcoreMesh(core_axis_name="c", subcore_axis_name="s")

    @functools.partial(
        pl.kernel,
        out_type=(
            jax.ShapeDtypeStruct((NC, NP, D), jnp.float32),
            jax.ShapeDtypeStruct((NC, NP, D), jnp.float32),
        ),
        mesh=mesh,
        compiler_params=pltpu.CompilerParams(use_tc_tiling_on_sc=False),
        scratch_types=[
            pltpu.VMEM((NB, KB), jnp.int32),
            pltpu.VMEM((NB, KB), jnp.int32),
            pltpu.VMEM((KB, D), jnp.float32),
            pltpu.VMEM((KB, D), jnp.float32),
            pltpu.VMEM_SHARED((NP, D), jnp.float32),
            pltpu.SemaphoreType.DMA,
            pltpu.SemaphoreType.DMA,
        ],
    )
    def spmm_kernel(row_hbm, col_hbm, mp_hbm, wp_hbm,
                    outm, outv, row_v, col_v, g0, g1,
                    acc, sem0, sem1):
        c = lax.axis_index("c")
        s = lax.axis_index("s")
        w = c * NS + s

        pltpu.sync_copy(row_hbm.at[w], row_v)
        pltpu.sync_copy(col_hbm.at[w], col_v)

        def one_pass(tbl_hbm, out_hbm):
            def zfill(j, carry):
                for k in range(D // 16):
                    g0[j, pl.ds(k * 16, 16)] = jnp.zeros((16,), jnp.float32)
                return carry

            lax.fori_loop(0, FC, zfill, 0)
            zsrc = g0.at[pl.ds(0, FC)]
            for k in range(RPT // FC):
                pltpu.sync_copy(zsrc, acc.at[pl.ds(s * RPT + k * FC, FC)])
            plsc.subcore_barrier()
            pltpu.async_copy(tbl_hbm.at[col_v.at[0]], g0, sem0)

            def body(i, carry):
                j0 = 2 * i
                j1 = j0 + 1
                pltpu.make_async_copy(tbl_hbm.at[col_v.at[j0]], g0, sem0).wait()
                pltpu.async_copy(tbl_hbm.at[col_v.at[j1]], g1, sem1)
                pltpu.sync_copy(g0, acc.at[row_v.at[j0]], add=True)
                pltpu.make_async_copy(tbl_hbm.at[col_v.at[j1]], g1, sem1).wait()

                @pl.when(j1 + 1 < NB)
                def _():
                    pltpu.async_copy(tbl_hbm.at[col_v.at[j1 + 1]], g0, sem0)

                pltpu.sync_copy(g1, acc.at[row_v.at[j1]], add=True)
                return carry

            lax.fori_loop(0, NB // 2, body, 0)
            plsc.subcore_barrier()
            fb = g1.at[pl.ds(0, FC)]
            for k in range(RPT // FC):
                sl = pl.ds(s * RPT + k * FC, FC)
                pltpu.sync_copy(acc.at[sl], fb)
                pltpu.sync_copy(fb, out_hbm.at[c, sl])
            plsc.subcore_barrier()

        one_pass(mp_hbm, outm)
        one_pass(wp_hbm, outv)

    return spmm_kernel


_spmm = _make_spmm()


def _tc1_body(x_ref, wm0_ref, bm0_ref, wv0_ref, bv0_ref, wm1_ref, bm1_ref,
              wv1_ref, bv1_ref, c0_ref, c1_ref,
              mp_ref, wp_ref, d0_ref, d1_ref):
    x = x_ref[...]
    m = jnp.dot(x, wm0_ref[...], preferred_element_type=jnp.float32) + bm0_ref[...]
    m = jnp.where(m > 0, m, (jnp.exp(m) - 1.0))
    m = jnp.dot(m, wm1_ref[...], preferred_element_type=jnp.float32) + bm1_ref[...]
    m = jnp.where(m > 0, m, (jnp.exp(m) - 1.0))
    v = jnp.maximum(jnp.dot(x, wv0_ref[...], preferred_element_type=jnp.float32) + bv0_ref[...], 0.0)
    v = jnp.maximum(jnp.dot(v, wv1_ref[...], preferred_element_type=jnp.float32) + bv1_ref[...], 0.0) + 1e-6
    att = jnp.exp(-v)
    m = m * att
    v = v * (att * att)
    deg = 1.0 + c0_ref[...] + c1_ref[...]
    d0 = lax.rsqrt(deg)
    d1 = 1.0 / deg
    mp_ref[...] = d0 * m
    wp_ref[...] = d1 * v
    d0_ref[...] = d0
    d1_ref[...] = d1


_tc1 = pl.pallas_call(
    _tc1_body,
    grid=(GRID,),
    in_specs=[
        pl.BlockSpec((BLK, D), lambda i: (i, 0)),
        pl.BlockSpec((D, D), lambda i: (0, 0)),
        pl.BlockSpec((1, D), lambda i: (0, 0)),
        pl.BlockSpec((D, D), lambda i: (0, 0)),
        pl.BlockSpec((1, D), lambda i: (0, 0)),
        pl.BlockSpec((D, D), lambda i: (0, 0)),
        pl.BlockSpec((1, D), lambda i: (0, 0)),
        pl.BlockSpec((D, D), lambda i: (0, 0)),
        pl.BlockSpec((1, D), lambda i: (0, 0)),
        pl.BlockSpec((BLK, 1), lambda i: (i, 0)),
        pl.BlockSpec((BLK, 1), lambda i: (i, 0)),
    ],
    out_specs=[
        pl.BlockSpec((BLK, D), lambda i: (i, 0)),
        pl.BlockSpec((BLK, D), lambda i: (i, 0)),
        pl.BlockSpec((BLK, 1), lambda i: (i, 0)),
        pl.BlockSpec((BLK, 1), lambda i: (i, 0)),
    ],
    out_shape=[
        jax.ShapeDtypeStruct((N, D), jnp.float32),
        jax.ShapeDtypeStruct((N, D), jnp.float32),
        jax.ShapeDtypeStruct((N, 1), jnp.float32),
        jax.ShapeDtypeStruct((N, 1), jnp.float32),
    ],
)


def _tc2_body(sm0_ref, sm1_ref, sv0_ref, sv1_ref, mp_ref, wp_ref,
              d0_ref, d1_ref, smp_ref, o_ref):
    mean = d0_ref[...] * (sm0_ref[...] + sm1_ref[...] + mp_ref[...])
    var = d1_ref[...] * (sv0_ref[...] + sv1_ref[...] + wp_ref[...])
    out = mean + smp_ref[...] * jnp.sqrt(var)
    mx = jnp.max(out, axis=1, keepdims=True)
    out = out - mx
    lse = jnp.log(jnp.sum(jnp.exp(out), axis=1, keepdims=True))
    o_ref[...] = out - lse


_tc2 = pl.pallas_call(
    _tc2_body,
    grid=(GRID,),
    in_specs=(
        [pl.BlockSpec((BLK, D), lambda i: (i, 0)) for _ in range(6)]
        + [pl.BlockSpec((BLK, 1), lambda i: (i, 0)) for _ in range(2)]
        + [pl.BlockSpec((BLK, D), lambda i: (i, 0))]
    ),
    out_specs=pl.BlockSpec((BLK, D), lambda i: (i, 0)),
    out_shape=jax.ShapeDtypeStruct((N, D), jnp.float32),
)


def kernel(x, edge_index, W_m0, b_m0, W_v0, b_v0, W_m1, b_m1, W_v1, b_v1):
    row = edge_index[0].astype(jnp.int32).reshape(NW, NB, KB)
    col = edge_index[1].astype(jnp.int32).reshape(NW, NB, KB)

    c0, c1 = _degree(row)
    mp, wp, d0, d1 = _tc1(
        x, W_m0, b_m0.reshape(1, D), W_v0, b_v0.reshape(1, D),
        W_m1, b_m1.reshape(1, D), W_v1, b_v1.reshape(1, D),
        c0[:N].reshape(N, 1), c1[:N].reshape(N, 1),
    )
    sm, sv = _spmm(row, col, mp, wp)
    sample = jax.random.normal(jax.random.key(42), (N, D), jnp.float32)
    return _tc2(sm[0, :N], sm[1, :N], sv[0, :N], sv[1, :N],
                mp, wp, d0, d1, sample)
